# baseline (speedup 1.0000x reference)
import functools

import jax
import jax.numpy as jnp
from jax import lax
from jax.experimental import pallas as pl
from jax.experimental.pallas import tpu as pltpu
from jax.experimental.pallas import tpu_sc as plsc

N = 10000
E = 320000
D = 128
NP = 10240
NS = 16
NW = 2 * NS
EPT = E // NW
CH = 128
NCHUNK = 80
CEPT = NCHUNK * CH
NSUP = NCHUNK // 4
NITER = NCHUNK // 8
RPT = NP // NS
BLK = 512
NB = NP // BLK

_mesh = plsc.VectorSubcoreMesh(core_axis_name="c", subcore_axis_name="s")


@functools.partial(
    pl.kernel,
    out_type=jax.ShapeDtypeStruct((2 * NP, D), jnp.float32),
    mesh=_mesh,
    scratch_types=(
        [pltpu.VMEM((4 * CH,), jnp.int32) for _ in range(2)]
        + [pltpu.VMEM((8, CH), jnp.int32) for _ in range(2)]
        + [pltpu.VMEM((CH, D), jnp.float32) for _ in range(2)]
        + [pltpu.VMEM_SHARED((NP, D), jnp.float32)]
        + [pltpu.SemaphoreType.DMA for _ in range(8)]
    ),
)
def _segsum_sc(y_hbm, srcp_hbm, dstp8_hbm, out_hbm, *refs):
    srcv = refs[0:2]
    dstv = refs[2:4]
    rows = refs[4:6]
    acc_sh = refs[6]
    gsem = refs[7:9]
    csem = refs[9:11]
    ssem = refs[11:13]
    dsem = refs[13:15]

    c = lax.axis_index("c")
    s = lax.axis_index("s")
    r0 = s * RPT
    w = c * NS + s
    ebase = w * CEPT
    w20 = w * NSUP

    def fire_super(sb, sidx):
        pltpu.async_copy(
            srcp_hbm.at[pl.ds(ebase + sidx * 4 * CH, 4 * CH)], srcv[sb], ssem[sb])
        pltpu.async_copy(dstp8_hbm.at[w20 + sidx], dstv[sb], dsem[sb])

    def wait_ssem(sb):
        pltpu.make_async_copy(
            srcp_hbm.at[pl.ds(ebase, 4 * CH)], srcv[sb], ssem[sb]).wait()

    def wait_dsem(sb):
        pltpu.make_async_copy(dstp8_hbm.at[w20], dstv[sb], dsem[sb]).wait()

    def fire_gather(rb, sb, slot):
        pltpu.async_copy(
            y_hbm.at[srcv[sb].at[pl.ds(slot * CH, CH)]], rows[rb], gsem[rb])

    def wait_gather(rb):
        pltpu.make_async_copy(
            y_hbm.at[srcv[0].at[pl.ds(0, CH)]], rows[rb], gsem[rb]).wait()

    def fire_scatter(rb, sb, slot):
        pltpu.async_copy(rows[rb], acc_sh.at[dstv[sb].at[slot]], csem[rb], add=True)

    def wait_scatter(rb):
        pltpu.make_async_copy(rows[rb], acc_sh.at[dstv[0].at[0]], csem[rb]).wait()

    fire_super(0, 0)
    pltpu.sync_copy(y_hbm.at[pl.ds(r0, RPT)], acc_sh.at[pl.ds(r0, RPT)])
    plsc.subcore_barrier()
    wait_ssem(0)
    wait_dsem(0)
    fire_gather(0, 0, 0)

    def body(i, carry):
        for u in range(8):
            b = u % 2
            slot = u % 4
            sb = u // 4
            wait_gather(b)
            fire_scatter(b, sb, slot)

            if u == 0:
                @pl.when(i > 0)
                def _():
                    wait_scatter(1)
            else:
                wait_scatter(1 - b)

            if u == 1:
                fire_super(1, 2 * i + 1)
            if u == 4:
                @pl.when(i < NITER - 1)
                def _():
                    fire_super(0, 2 * i + 2)

            if u == 3:
                wait_ssem(1)
                wait_dsem(1)
                fire_gather(0, 1, 0)
            elif u == 7:
                @pl.when(i < NITER - 1)
                def _():
                    wait_ssem(0)
                    wait_dsem(0)
                    fire_gather(0, 0, 0)
            else:
                fire_gather(1 - b, (u + 1) // 4, (u + 1) % 4)
        return carry

    lax.fori_loop(0, NITER, body, 0)
    wait_scatter(1)
    plsc.subcore_barrier()
    pltpu.sync_copy(acc_sh.at[pl.ds(r0, RPT)], out_hbm.at[pl.ds(c * NP + r0, RPT)])


def _mm_body(x_ref, w_ref, o_ref):
    o_ref[...] = lax.dot_general(
        x_ref[...], w_ref[...], (((1,), (1,)), ((), ())),
        preferred_element_type=jnp.float32)


def _relu_mm_body(sa_ref, sb_ref, y_ref, b_ref, w_ref, o_ref):
    h = jnp.maximum(sa_ref[...] + sb_ref[...] - y_ref[...] + b_ref[...], 0.0)
    o_ref[...] = lax.dot_general(
        h, w_ref[...], (((1,), (1,)), ((), ())),
        preferred_element_type=jnp.float32)


def _final_body(sa_ref, sb_ref, y_ref, b_ref, o_ref):
    o_ref[...] = sa_ref[...] + sb_ref[...] - y_ref[...] + b_ref[...]


def kernel(x, adj, W1, b1, W2, b2):
    srcp = jnp.pad(adj[0].reshape(NW, EPT),
                   ((0, 0), (0, CEPT - EPT))).reshape(-1)
    dstp8 = jnp.pad(
        jnp.pad(adj[1].reshape(NW, EPT), ((0, 0), (0, CEPT - EPT)),
                constant_values=NP - 1).reshape(NW, NSUP, 4, CH),
        ((0, 0), (0, 0), (0, 4), (0, 0))).reshape(NW * NSUP, 8, CH)
    x_pad = jnp.pad(x, ((0, NP - N), (0, 0)))

    y1 = pl.pallas_call(
        _mm_body,
        grid=(NB,),
        in_specs=[
            pl.BlockSpec((BLK, D), lambda j: (j, 0)),
            pl.BlockSpec((D, D), lambda j: (0, 0)),
        ],
        out_specs=pl.BlockSpec((BLK, D), lambda j: (j, 0)),
        out_shape=jax.ShapeDtypeStruct((NP, D), jnp.float32),
    )(x_pad, W1)

    s1 = _segsum_sc(y1, srcp, dstp8)

    y2 = pl.pallas_call(
        _relu_mm_body,
        grid=(NB,),
        in_specs=[
            pl.BlockSpec((BLK, D), lambda j: (j, 0)),
            pl.BlockSpec((BLK, D), lambda j: (NB + j, 0)),
            pl.BlockSpec((BLK, D), lambda j: (j, 0)),
            pl.BlockSpec((1, D), lambda j: (0, 0)),
            pl.BlockSpec((D, D), lambda j: (0, 0)),
        ],
        out_specs=pl.BlockSpec((BLK, D), lambda j: (j, 0)),
        out_shape=jax.ShapeDtypeStruct((NP, D), jnp.float32),
    )(s1, s1, y1, b1.reshape(1, D), W2)

    s2 = _segsum_sc(y2, srcp, dstp8)

    out = pl.pallas_call(
        _final_body,
        grid=(NB,),
        in_specs=[
            pl.BlockSpec((BLK, D), lambda j: (j, 0)),
            pl.BlockSpec((BLK, D), lambda j: (NB + j, 0)),
            pl.BlockSpec((BLK, D), lambda j: (j, 0)),
            pl.BlockSpec((1, D), lambda j: (0, 0)),
        ],
        out_specs=pl.BlockSpec((BLK, D), lambda j: (j, 0)),
        out_shape=jax.ShapeDtypeStruct((NP, D), jnp.float32),
    )(s2, s2, y2, b2.reshape(1, D))

    return out[:N]


# speedup vs baseline: 1.4123x; 1.4123x over previous
import functools

import jax
import jax.numpy as jnp
from jax import lax
from jax.experimental import pallas as pl
from jax.experimental.pallas import tpu as pltpu
from jax.experimental.pallas import tpu_sc as plsc

N = 10000
E = 320000
D = 128
NP = 10240
NS = 16
NW = 2 * NS
EPT = E // NW
CH = 80
NCHUNK = 125
RPT = NP // NS
BLK = 512
NB = NP // BLK

_mesh = plsc.VectorSubcoreMesh(core_axis_name="c", subcore_axis_name="s")


@functools.partial(
    pl.kernel,
    out_type=jax.ShapeDtypeStruct((2 * NP, D), jnp.float32),
    mesh=_mesh,
    scratch_types=[
        pltpu.VMEM((8, CH), jnp.int32),
        pltpu.VMEM((CH, D), jnp.float32),
        pltpu.VMEM_SHARED((NP, D), jnp.float32),
        pltpu.SemaphoreType.DMA,
    ],
)
def _segsum_sc(y_hbm, edges_hbm, out_hbm, idx_v, rows_v, acc_sh, sem):
    c = lax.axis_index("c")
    s = lax.axis_index("s")
    r0 = s * RPT
    kbase = (c * NS + s) * NCHUNK
    pltpu.sync_copy(y_hbm.at[pl.ds(r0, RPT)], acc_sh.at[pl.ds(r0, RPT)])
    plsc.subcore_barrier()

    def body(k, carry):
        pltpu.sync_copy(edges_hbm.at[kbase + k], idx_v)
        pltpu.async_copy(y_hbm.at[idx_v.at[0]], rows_v, sem).wait()
        pltpu.sync_copy(rows_v, acc_sh.at[idx_v.at[1]], add=True)
        return carry

    lax.fori_loop(0, NCHUNK, body, 0)
    plsc.subcore_barrier()
    pltpu.sync_copy(acc_sh.at[pl.ds(r0, RPT)], out_hbm.at[pl.ds(c * NP + r0, RPT)])


def _mm_body(x_ref, w_ref, o_ref):
    o_ref[...] = lax.dot_general(
        x_ref[...], w_ref[...], (((1,), (1,)), ((), ())),
        preferred_element_type=jnp.float32)


def _relu_mm_body(sa_ref, sb_ref, y_ref, b_ref, w_ref, o_ref):
    h = jnp.maximum(sa_ref[...] + sb_ref[...] - y_ref[...] + b_ref[...], 0.0)
    o_ref[...] = lax.dot_general(
        h, w_ref[...], (((1,), (1,)), ((), ())),
        preferred_element_type=jnp.float32)


def _final_body(sa_ref, sb_ref, y_ref, b_ref, o_ref):
    o_ref[...] = sa_ref[...] + sb_ref[...] - y_ref[...] + b_ref[...]


def kernel(x, adj, W1, b1, W2, b2):
    edges8 = jnp.pad(
        jnp.stack([adj[0].reshape(NW, NCHUNK, CH),
                   adj[1].reshape(NW, NCHUNK, CH)], axis=2),
        ((0, 0), (0, 0), (0, 6), (0, 0))).reshape(NW * NCHUNK, 8, CH)
    x_pad = jnp.pad(x, ((0, NP - N), (0, 0)))

    y1 = pl.pallas_call(
        _mm_body,
        grid=(NB,),
        in_specs=[
            pl.BlockSpec((BLK, D), lambda j: (j, 0)),
            pl.BlockSpec((D, D), lambda j: (0, 0)),
        ],
        out_specs=pl.BlockSpec((BLK, D), lambda j: (j, 0)),
        out_shape=jax.ShapeDtypeStruct((NP, D), jnp.float32),
    )(x_pad, W1)

    s1 = _segsum_sc(y1, edges8)

    y2 = pl.pallas_call(
        _relu_mm_body,
        grid=(NB,),
        in_specs=[
            pl.BlockSpec((BLK, D), lambda j: (j, 0)),
            pl.BlockSpec((BLK, D), lambda j: (NB + j, 0)),
            pl.BlockSpec((BLK, D), lambda j: (j, 0)),
            pl.BlockSpec((1, D), lambda j: (0, 0)),
            pl.BlockSpec((D, D), lambda j: (0, 0)),
        ],
        out_specs=pl.BlockSpec((BLK, D), lambda j: (j, 0)),
        out_shape=jax.ShapeDtypeStruct((NP, D), jnp.float32),
    )(s1, s1, y1, b1.reshape(1, D), W2)

    s2 = _segsum_sc(y2, edges8)

    out = pl.pallas_call(
        _final_body,
        grid=(NB,),
        in_specs=[
            pl.BlockSpec((BLK, D), lambda j: (j, 0)),
            pl.BlockSpec((BLK, D), lambda j: (NB + j, 0)),
            pl.BlockSpec((BLK, D), lambda j: (j, 0)),
            pl.BlockSpec((1, D), lambda j: (0, 0)),
        ],
        out_specs=pl.BlockSpec((BLK, D), lambda j: (j, 0)),
        out_shape=jax.ShapeDtypeStruct((NP, D), jnp.float32),
    )(s2, s2, y2, b2.reshape(1, D))

    return out[:N]
